# Initial kernel scaffold; baseline (speedup 1.0000x reference)
import jax
import jax.numpy as jnp
from jax.experimental import pallas as pl


def kernel(featmap_s0, featmap_s1, gt_s0, gt_s1):
    raise NotImplementedError("write your pallas kernel here")


# speedup vs baseline: 9.6751x; 9.6751x over previous
import functools

import jax
import jax.numpy as jnp
from jax import lax
from jax.experimental import pallas as pl
from jax.experimental.pallas import tpu as pltpu
from jax.experimental.pallas import tpu_sc as plsc

F32 = jnp.float32
I32 = jnp.int32

B = 8
S0 = 512 * 512
S1 = 256 * 256
LBL = 32
LANES = 16
NTILES = 32
TPB = NTILES // B
BLK = 8192
NB0 = S0 // (TPB * BLK)
NB1 = S1 // (TPB * BLK)

MVAR = 0.1
MDIST = 1.5
VAR_W = 1.0
DIST_W = 1.0

_MESH = plsc.VectorSubcoreMesh(core_axis_name="c", subcore_axis_name="s")
_SC_PARAMS = pltpu.CompilerParams(needs_layout_passes=False)


def _wid():
    return lax.axis_index("s") * 2 + lax.axis_index("c")


@functools.partial(
    pl.kernel,
    out_type=jax.ShapeDtypeStruct((NTILES, 4, LBL), F32),
    mesh=_MESH,
    compiler_params=_SC_PARAMS,
    scratch_types=[
        pltpu.VMEM((BLK,), F32),
        pltpu.VMEM((BLK,), I32),
        pltpu.VMEM((LBL * LANES,), F32),
        pltpu.VMEM((LBL * LANES,), F32),
        pltpu.VMEM((LBL * LANES,), F32),
        pltpu.VMEM((LBL * LANES,), F32),
        pltpu.VMEM((LBL,), F32),
    ],
)
def _sc_bins(f0, g0, f1, g1, out, fbuf, gbuf, cnt0, sum0, cnt1, sum1, obuf):
    wid = _wid()
    b = wid // TPB
    q = wid % TPB
    lane = lax.iota(I32, LANES)
    ones = jnp.ones((LANES,), F32)
    zeros = jnp.zeros((LANES,), F32)

    for r in range(LBL):
        o = r * LANES
        cnt0[pl.ds(o, LANES)] = zeros
        sum0[pl.ds(o, LANES)] = zeros
        cnt1[pl.ds(o, LANES)] = zeros
        sum1[pl.ds(o, LANES)] = zeros

    def make_body(cnt, ssum):
        def body(i, _):
            o = pl.multiple_of(i * LANES, LANES)
            fv = fbuf[pl.ds(o, LANES)]
            gv = gbuf[pl.ds(o, LANES)]
            idx = gv * LANES + lane
            plsc.addupdate_scatter(ssum, [idx], fv)
            plsc.addupdate_scatter(cnt, [idx], ones)
            return 0
        return body

    body0 = make_body(cnt0, sum0)
    body1 = make_body(cnt1, sum1)

    for blk in range(NB0):
        off = q * (S0 // TPB) + blk * BLK
        pltpu.sync_copy(f0.at[b, pl.ds(off, BLK)], fbuf)
        pltpu.sync_copy(g0.at[b, pl.ds(off, BLK)], gbuf)
        lax.fori_loop(0, BLK // LANES, body0, 0)
    for blk in range(NB1):
        off = q * (S1 // TPB) + blk * BLK
        pltpu.sync_copy(f1.at[b, pl.ds(off, BLK)], fbuf)
        pltpu.sync_copy(g1.at[b, pl.ds(off, BLK)], gbuf)
        lax.fori_loop(0, BLK // LANES, body1, 0)

    for k, ref in enumerate((cnt0, sum0, cnt1, sum1)):
        for half in range(LBL // LANES):
            acc = zeros
            base = (lane + half * LANES) * LANES
            for j in range(LANES):
                acc = acc + plsc.load_gather(ref, [base + j])
            obuf[pl.ds(half * LANES, LANES)] = acc
        pltpu.sync_copy(obuf, out.at[wid, k])


def _tc_tables_body(bins_ref, tbl_ref):
    xl = bins_ref[...]
    ys = [jnp.sum(xl[b * TPB:(b + 1) * TPB], axis=0) for b in range(B)]
    c0 = jnp.concatenate([y[0:1] for y in ys], axis=0)
    s0 = jnp.concatenate([y[1:2] for y in ys], axis=0)
    c1 = jnp.concatenate([y[2:3] for y in ys], axis=0)
    s1 = jnp.concatenate([y[3:4] for y in ys], axis=0)

    total = c0 + c1
    mean = (s0 + s1) / jnp.maximum(total, 1.0)
    lbl = lax.broadcasted_iota(I32, (B, LBL), 1)
    labmask = (lbl >= 1) & (lbl <= 16)
    cmax = jnp.max(jnp.where((c0 > 0) & labmask, lbl, 0))
    valid = (total > 0) & (lbl <= cmax) & labmask
    w0 = jnp.where(valid & (c0 > 0), 1.0 / jnp.maximum(c0, 1.0), 0.0)
    w1 = jnp.where(valid & (c1 > 0), 1.0 / jnp.maximum(c1, 1.0), 0.0)
    pull_count = jnp.sum(valid.astype(F32))

    vf = valid.astype(F32)
    mi = lax.broadcast_in_dim(mean, (B, LBL, LBL), (0, 1))
    mj = lax.broadcast_in_dim(mean, (B, LBL, LBL), (0, 2))
    vi = lax.broadcast_in_dim(vf, (B, LBL, LBL), (0, 1))
    vj = lax.broadcast_in_dim(vf, (B, LBL, LBL), (0, 2))
    ii = lax.broadcasted_iota(I32, (B, LBL, LBL), 1)
    jj = lax.broadcasted_iota(I32, (B, LBL, LBL), 2)
    pv = vi * vj * (ii != jj).astype(F32)
    il = jnp.maximum(2.0 * MDIST - jnp.abs(mi - mj), 0.0) ** 2
    push_sum = jnp.sum(il * pv)
    push_count = jnp.sum(pv)
    push_loss = jnp.where(push_count > 0, push_sum / push_count * DIST_W, 0.0)

    ir = lax.broadcasted_iota(I32, (1, LBL), 1)
    misc = jnp.where(ir == 0, push_loss, jnp.where(ir == 1, pull_count, 0.0))
    tbl_ref[0] = mean
    tbl_ref[1] = w0
    tbl_ref[2] = w1
    tbl_ref[3] = jnp.broadcast_to(misc, (B, LBL))


_tc_tables = pl.pallas_call(
    _tc_tables_body,
    out_shape=jax.ShapeDtypeStruct((4, B, LBL), F32),
)


@functools.partial(
    pl.kernel,
    out_type=jax.ShapeDtypeStruct((NTILES, LANES), F32),
    mesh=_MESH,
    compiler_params=_SC_PARAMS,
    scratch_types=[
        pltpu.VMEM((BLK,), F32),
        pltpu.VMEM((BLK,), I32),
        pltpu.VMEM((LBL,), F32),
        pltpu.VMEM((LBL,), F32),
        pltpu.VMEM((LBL,), F32),
        pltpu.VMEM((LANES,), F32),
    ],
)
def _sc_pull(f0, g0, f1, g1, tbl, out, fbuf, gbuf, mv, w0v, w1v, accbuf):
    wid = _wid()
    b = wid // TPB
    q = wid % TPB

    pltpu.sync_copy(tbl.at[0, b], mv)
    pltpu.sync_copy(tbl.at[1, b], w0v)
    pltpu.sync_copy(tbl.at[2, b], w1v)

    def make_body(wv):
        def body(i, acc):
            o = pl.multiple_of(i * LANES, LANES)
            fv = fbuf[pl.ds(o, LANES)]
            gv = gbuf[pl.ds(o, LANES)]
            m = plsc.load_gather(mv, [gv])
            w = plsc.load_gather(wv, [gv])
            t = jnp.maximum(jnp.abs(fv - m) - MVAR, 0.0)
            return acc + t * t * w
        return body

    body0 = make_body(w0v)
    body1 = make_body(w1v)

    acc = jnp.zeros((LANES,), F32)
    for blk in range(NB0):
        off = q * (S0 // TPB) + blk * BLK
        pltpu.sync_copy(f0.at[b, pl.ds(off, BLK)], fbuf)
        pltpu.sync_copy(g0.at[b, pl.ds(off, BLK)], gbuf)
        acc = lax.fori_loop(0, BLK // LANES, body0, acc)
    for blk in range(NB1):
        off = q * (S1 // TPB) + blk * BLK
        pltpu.sync_copy(f1.at[b, pl.ds(off, BLK)], fbuf)
        pltpu.sync_copy(g1.at[b, pl.ds(off, BLK)], gbuf)
        acc = lax.fori_loop(0, BLK // LANES, body1, acc)

    accbuf[...] = acc
    pltpu.sync_copy(accbuf, out.at[wid])


def _tc_final_body(part_ref, tbl_ref, out_ref):
    ps = jnp.sum(part_ref[...])
    row = tbl_ref[3, 0:1, :]
    ir = lax.broadcasted_iota(I32, (1, LBL), 1)
    push_loss = jnp.sum(jnp.where(ir == 0, row, 0.0))
    pc = jnp.sum(jnp.where(ir == 1, row, 0.0))
    pull = jnp.where(pc > 0, ps / pc * VAR_W, 0.0)
    out_ref[...] = jnp.reshape(push_loss + pull, (1, 1))


_tc_final = pl.pallas_call(
    _tc_final_body,
    out_shape=jax.ShapeDtypeStruct((1, 1), F32),
)


@jax.jit
def kernel(featmap_s0, featmap_s1, gt_s0, gt_s1):
    f0 = featmap_s0.reshape(B, S0).astype(F32)
    f1 = featmap_s1.reshape(B, S1).astype(F32)
    g0 = gt_s0.reshape(B, S0).astype(I32)
    g1 = gt_s1.reshape(B, S1).astype(I32)

    bins = _sc_bins(f0, g0, f1, g1)
    tbl = _tc_tables(bins)
    partials = _sc_pull(f0, g0, f1, g1, tbl)
    out = _tc_final(partials, tbl)
    return out.reshape(())
